# baseline (speedup 1.0000x reference)
import functools
import math

import jax
import jax.numpy as jnp
from jax import lax
from jax.experimental import pallas as pl
from jax.experimental.pallas import tpu as pltpu
from jax.experimental.pallas import tpu_sc as plsc

N = 10000
E = 320000
IN_CH = 128
HID = 64
OUT_CH = 128
L_LAYERS = 64
ALPHA = 0.5
THETA = 1.0

NC = 2
NS = 16
CH = 128
NCHUNK = 84
NBUF = 4
EPT = NCHUNK * CH
NPAD = 10240
TRASH = N
ROWS_PT = NPAD // NS
ZB = ROWS_PT // CH


def _sc_agg_body(u_hbm, src_hbm, dst_hbm, out_hbm, src_v, dst_v,
                 b0, b1, b2, b3, agg_sh, s0, s1, s2, s3):
    bufs = (b0, b1, b2, b3)
    sems = (s0, s1, s2, s3)
    cid = lax.axis_index("c")
    sid = lax.axis_index("s")
    pltpu.sync_copy(src_hbm.at[cid, sid], src_v)
    pltpu.sync_copy(dst_hbm.at[cid, sid], dst_v)

    def _zrow(r, carry):
        def _zcol(k, c2):
            b0[r, pl.ds(k * 16, 16)] = jnp.zeros((16,), jnp.float32)
            return c2
        return lax.fori_loop(0, HID // 16, _zcol, carry)
    lax.fori_loop(0, CH, _zrow, 0)
    base = sid * ROWS_PT

    def _zcopy(k, carry):
        pltpu.sync_copy(b0, agg_sh.at[pl.ds(base + k * CH, CH)])
        return carry
    lax.fori_loop(0, ZB, _zcopy, 0)
    plsc.subcore_barrier()

    b0, b1 = bufs[0], bufs[1]
    s0g, s1g = sems[0], sems[1]
    pltpu.async_copy(u_hbm.at[src_v.at[0]], b0, s0g)

    def _pair(p, carry):
        c0 = 2 * p
        pltpu.make_async_copy(u_hbm.at[src_v.at[c0]], b0, s0g).wait()
        pltpu.async_copy(u_hbm.at[src_v.at[c0 + 1]], b1, s1g)
        pltpu.sync_copy(b0, agg_sh.at[dst_v.at[c0]], add=True)
        pltpu.make_async_copy(u_hbm.at[src_v.at[c0 + 1]], b1, s1g).wait()
        pltpu.async_copy(u_hbm.at[src_v.at[c0 + 2]], b0, s0g)
        pltpu.sync_copy(b1, agg_sh.at[dst_v.at[c0 + 1]], add=True)
        return carry
    lax.fori_loop(0, NCHUNK // 2 - 1, _pair, 0)
    c0 = NCHUNK - 2
    pltpu.make_async_copy(u_hbm.at[src_v.at[c0]], b0, s0g).wait()
    pltpu.async_copy(u_hbm.at[src_v.at[c0 + 1]], b1, s1g)
    pltpu.sync_copy(b0, agg_sh.at[dst_v.at[c0]], add=True)
    pltpu.make_async_copy(u_hbm.at[src_v.at[c0 + 1]], b1, s1g).wait()
    pltpu.sync_copy(b1, agg_sh.at[dst_v.at[c0 + 1]], add=True)
    plsc.subcore_barrier()

    def _ocopy(k, carry):
        sl = pl.ds(base + k * CH, CH)
        pltpu.sync_copy(agg_sh.at[sl], out_hbm.at[cid, sl])
        return carry
    lax.fori_loop(0, ZB, _ocopy, 0)


@jax.jit
def _sc_agg(u, src_idx, dst_idx):
    return pl.kernel(
        _sc_agg_body,
        out_type=jax.ShapeDtypeStruct((NC, NPAD, HID), jnp.float32),
        mesh=plsc.VectorSubcoreMesh(core_axis_name="c", subcore_axis_name="s"),
        scratch_types=[
            pltpu.VMEM((NCHUNK, CH), jnp.int32),
            pltpu.VMEM((NCHUNK, CH), jnp.int32),
            pltpu.VMEM((CH, HID), jnp.float32),
            pltpu.VMEM((CH, HID), jnp.float32),
            pltpu.VMEM((CH, HID), jnp.float32),
            pltpu.VMEM((CH, HID), jnp.float32),
            pltpu.VMEM_SHARED((NPAD, HID), jnp.float32),
            pltpu.SemaphoreType.DMA,
            pltpu.SemaphoreType.DMA,
            pltpu.SemaphoreType.DMA,
            pltpu.SemaphoreType.DMA,
        ],
        compiler_params=pltpu.CompilerParams(use_tc_tiling_on_sc=False),
    )(u, src_idx, dst_idx)


def _lin1_body(x_ref, w1_ref, b1_ref, pdeg_ref, h0_ref, u0_ref, dinv_ref):
    deg = pdeg_ref[0, :, 0:1] + pdeg_ref[1, :, 0:1]
    dinv = jnp.where(deg > 0, lax.rsqrt(deg), 0.0)
    h0 = jnp.maximum(
        jnp.dot(x_ref[...], w1_ref[...], preferred_element_type=jnp.float32)
        + b1_ref[...], 0.0)
    h0_ref[...] = h0
    u0_ref[...] = dinv * h0
    dinv_ref[...] = dinv


def _tc_layer_body(p_ref, h0_ref, dinv_ref, wp_ref, u_ref, h_ref):
    dinv = dinv_ref[...]
    p = p_ref[0] + p_ref[1]
    t = (1.0 - ALPHA) * (dinv * p) + ALPHA * h0_ref[...]
    h = jnp.maximum(
        jnp.dot(t, wp_ref[...], preferred_element_type=jnp.float32), 0.0)
    h_ref[...] = h
    u_ref[...] = dinv * h


def _final_body(h_ref, w2_ref, b2_ref, o_ref):
    z = (jnp.dot(h_ref[...], w2_ref[...], preferred_element_type=jnp.float32)
         + b2_ref[...])
    m = jnp.max(z, axis=1, keepdims=True)
    lse = jnp.log(jnp.sum(jnp.exp(z - m), axis=1, keepdims=True)) + m
    o_ref[...] = z - lse


_RB = 2048


@jax.jit
def _lin1(xp, w1, b1, pdeg):
    grid = (NPAD // _RB,)
    return pl.pallas_call(
        _lin1_body,
        grid=grid,
        in_specs=[
            pl.BlockSpec((_RB, IN_CH), lambda i: (i, 0)),
            pl.BlockSpec((IN_CH, HID), lambda i: (0, 0)),
            pl.BlockSpec((1, HID), lambda i: (0, 0)),
            pl.BlockSpec((NC, _RB, HID), lambda i: (0, i, 0)),
        ],
        out_specs=[
            pl.BlockSpec((_RB, HID), lambda i: (i, 0)),
            pl.BlockSpec((_RB, HID), lambda i: (i, 0)),
            pl.BlockSpec((_RB, 1), lambda i: (i, 0)),
        ],
        out_shape=[
            jax.ShapeDtypeStruct((NPAD, HID), jnp.float32),
            jax.ShapeDtypeStruct((NPAD, HID), jnp.float32),
            jax.ShapeDtypeStruct((NPAD, 1), jnp.float32),
        ],
    )(xp, w1, b1, pdeg)


@jax.jit
def _tc_layer(part, h0, dinv, wp):
    grid = (NPAD // _RB,)
    return pl.pallas_call(
        _tc_layer_body,
        grid=grid,
        in_specs=[
            pl.BlockSpec((NC, _RB, HID), lambda i: (0, i, 0)),
            pl.BlockSpec((_RB, HID), lambda i: (i, 0)),
            pl.BlockSpec((_RB, 1), lambda i: (i, 0)),
            pl.BlockSpec((HID, HID), lambda i: (0, 0)),
        ],
        out_specs=[
            pl.BlockSpec((_RB, HID), lambda i: (i, 0)),
            pl.BlockSpec((_RB, HID), lambda i: (i, 0)),
        ],
        out_shape=[
            jax.ShapeDtypeStruct((NPAD, HID), jnp.float32),
            jax.ShapeDtypeStruct((NPAD, HID), jnp.float32),
        ],
    )(part, h0, dinv, wp)


_FB = 2000


@jax.jit
def _final(h, w2, b2):
    grid = (N // _FB,)
    return pl.pallas_call(
        _final_body,
        grid=grid,
        in_specs=[
            pl.BlockSpec((_FB, HID), lambda i: (i, 0)),
            pl.BlockSpec((HID, OUT_CH), lambda i: (0, 0)),
            pl.BlockSpec((1, OUT_CH), lambda i: (0, 0)),
        ],
        out_specs=pl.BlockSpec((_FB, OUT_CH), lambda i: (i, 0)),
        out_shape=jax.ShapeDtypeStruct((N, OUT_CH), jnp.float32),
    )(h, w2, b2)


def kernel(x, edge_index, lin1_w, lin1_b, conv_w, lin2_w, lin2_b):
    ei = edge_index.astype(jnp.int32)
    loops = jnp.arange(N, dtype=jnp.int32)
    src = jnp.concatenate([ei[0], loops])
    dst = jnp.concatenate([ei[1], loops])
    total = NC * NS * EPT
    padlen = total - src.shape[0]
    src = jnp.concatenate([src, jnp.zeros((padlen,), jnp.int32)])
    dst = jnp.concatenate([dst, jnp.full((padlen,), TRASH, jnp.int32)])
    src_idx = src.reshape(NC, NS, NCHUNK, CH)
    dst_idx = dst.reshape(NC, NS, NCHUNK, CH)

    beta = jnp.log(THETA / (jnp.arange(1, L_LAYERS + 1, dtype=jnp.float32)) + 1.0)
    wp_all = (beta[:, None, None] * conv_w
              + (1.0 - beta)[:, None, None] * jnp.eye(HID, dtype=jnp.float32))

    xp = jnp.pad(x, ((0, NPAD - N), (0, 0)))
    ones_u = jnp.ones((NPAD, HID), jnp.float32)
    pdeg = _sc_agg(ones_u, src_idx, dst_idx)
    h0, u0, dinv = _lin1(xp, lin1_w, lin1_b.reshape(1, HID), pdeg)

    def body(i, carry):
        u, _h = carry
        part = _sc_agg(u, src_idx, dst_idx)
        wp = lax.dynamic_index_in_dim(wp_all, i, keepdims=False)
        u, h = _tc_layer(part, h0, dinv, wp)
        return (u, h)

    _u, h = lax.fori_loop(0, L_LAYERS, body, (u0, h0))
    return _final(h, lin2_w, lin2_b.reshape(1, OUT_CH))


# speedup vs baseline: 1.0331x; 1.0331x over previous
import functools
import math

import jax
import jax.numpy as jnp
from jax import lax
from jax.experimental import pallas as pl
from jax.experimental.pallas import tpu as pltpu
from jax.experimental.pallas import tpu_sc as plsc

N = 10000
E = 320000
IN_CH = 128
HID = 64
OUT_CH = 128
L_LAYERS = 64
ALPHA = 0.5
THETA = 1.0

NC = 2
NS = 16
CH = 128
G = 4
NCHUNK = 84
QC = NCHUNK // G
EPT = NCHUNK * CH
NPAD = 10240
TRASH = N
ROWS_PT = NPAD // NS
ZB = ROWS_PT // CH


def _sc_agg_body(u_hbm, src_hbm, dst_hbm, out_hbm, src_v, dst_v, buf, agg_sh, sem):
    cid = lax.axis_index("c")
    sid = lax.axis_index("s")
    pltpu.sync_copy(src_hbm.at[cid, sid], src_v)
    pltpu.sync_copy(dst_hbm.at[cid, sid], dst_v)

    def _zrow(r, carry):
        def _zcol(k, c2):
            buf[r, pl.ds(k * 16, 16)] = jnp.zeros((16,), jnp.float32)
            return c2
        return lax.fori_loop(0, HID // 16, _zcol, carry)
    lax.fori_loop(0, CH, _zrow, 0)
    base = sid * ROWS_PT

    def _zcopy(k, carry):
        pltpu.sync_copy(buf.at[pl.ds(0, CH)], agg_sh.at[pl.ds(base + k * CH, CH)])
        return carry
    lax.fori_loop(0, ZB, _zcopy, 0)
    plsc.subcore_barrier()

    def _chunk(q, carry):
        pltpu.async_copy(u_hbm.at[src_v.at[q]], buf, sem).wait()
        pltpu.sync_copy(buf, agg_sh.at[dst_v.at[q]], add=True)
        return carry
    lax.fori_loop(0, QC, _chunk, 0)
    plsc.subcore_barrier()

    def _ocopy(k, carry):
        sl = pl.ds(base + k * CH, CH)
        pltpu.sync_copy(agg_sh.at[sl], out_hbm.at[cid, sl])
        return carry
    lax.fori_loop(0, ZB, _ocopy, 0)


@jax.jit
def _sc_agg(u, src_idx, dst_idx):
    return pl.kernel(
        _sc_agg_body,
        out_type=jax.ShapeDtypeStruct((NC, NPAD, HID), jnp.float32),
        mesh=plsc.VectorSubcoreMesh(core_axis_name="c", subcore_axis_name="s"),
        scratch_types=[
            pltpu.VMEM((QC, G * CH), jnp.int32),
            pltpu.VMEM((QC, G * CH), jnp.int32),
            pltpu.VMEM((G * CH, HID), jnp.float32),
            pltpu.VMEM_SHARED((NPAD, HID), jnp.float32),
            pltpu.SemaphoreType.DMA,
        ],
        compiler_params=pltpu.CompilerParams(use_tc_tiling_on_sc=False),
    )(u, src_idx, dst_idx)


def _lin1_body(x_ref, w1_ref, b1_ref, pdeg_ref, h0_ref, u0_ref, dinv_ref):
    deg = pdeg_ref[0, :, 0:1] + pdeg_ref[1, :, 0:1]
    dinv = jnp.where(deg > 0, lax.rsqrt(deg), 0.0)
    h0 = jnp.maximum(
        jnp.dot(x_ref[...], w1_ref[...], preferred_element_type=jnp.float32)
        + b1_ref[...], 0.0)
    h0_ref[...] = h0
    u0_ref[...] = dinv * h0
    dinv_ref[...] = dinv


def _tc_layer_body(p_ref, h0_ref, dinv_ref, wp_ref, u_ref, h_ref):
    dinv = dinv_ref[...]
    p = p_ref[0] + p_ref[1]
    t = (1.0 - ALPHA) * (dinv * p) + ALPHA * h0_ref[...]
    h = jnp.maximum(
        jnp.dot(t, wp_ref[...], preferred_element_type=jnp.float32), 0.0)
    h_ref[...] = h
    u_ref[...] = dinv * h


def _final_body(h_ref, w2_ref, b2_ref, o_ref):
    z = (jnp.dot(h_ref[...], w2_ref[...], preferred_element_type=jnp.float32)
         + b2_ref[...])
    m = jnp.max(z, axis=1, keepdims=True)
    lse = jnp.log(jnp.sum(jnp.exp(z - m), axis=1, keepdims=True)) + m
    o_ref[...] = z - lse


_RB = 2048


@jax.jit
def _lin1(xp, w1, b1, pdeg):
    grid = (NPAD // _RB,)
    return pl.pallas_call(
        _lin1_body,
        grid=grid,
        in_specs=[
            pl.BlockSpec((_RB, IN_CH), lambda i: (i, 0)),
            pl.BlockSpec((IN_CH, HID), lambda i: (0, 0)),
            pl.BlockSpec((1, HID), lambda i: (0, 0)),
            pl.BlockSpec((NC, _RB, HID), lambda i: (0, i, 0)),
        ],
        out_specs=[
            pl.BlockSpec((_RB, HID), lambda i: (i, 0)),
            pl.BlockSpec((_RB, HID), lambda i: (i, 0)),
            pl.BlockSpec((_RB, 1), lambda i: (i, 0)),
        ],
        out_shape=[
            jax.ShapeDtypeStruct((NPAD, HID), jnp.float32),
            jax.ShapeDtypeStruct((NPAD, HID), jnp.float32),
            jax.ShapeDtypeStruct((NPAD, 1), jnp.float32),
        ],
    )(xp, w1, b1, pdeg)


@jax.jit
def _tc_layer(part, h0, dinv, wp):
    grid = (NPAD // _RB,)
    return pl.pallas_call(
        _tc_layer_body,
        grid=grid,
        in_specs=[
            pl.BlockSpec((NC, _RB, HID), lambda i: (0, i, 0)),
            pl.BlockSpec((_RB, HID), lambda i: (i, 0)),
            pl.BlockSpec((_RB, 1), lambda i: (i, 0)),
            pl.BlockSpec((HID, HID), lambda i: (0, 0)),
        ],
        out_specs=[
            pl.BlockSpec((_RB, HID), lambda i: (i, 0)),
            pl.BlockSpec((_RB, HID), lambda i: (i, 0)),
        ],
        out_shape=[
            jax.ShapeDtypeStruct((NPAD, HID), jnp.float32),
            jax.ShapeDtypeStruct((NPAD, HID), jnp.float32),
        ],
    )(part, h0, dinv, wp)


_FB = 2000


@jax.jit
def _final(h, w2, b2):
    grid = (N // _FB,)
    return pl.pallas_call(
        _final_body,
        grid=grid,
        in_specs=[
            pl.BlockSpec((_FB, HID), lambda i: (i, 0)),
            pl.BlockSpec((HID, OUT_CH), lambda i: (0, 0)),
            pl.BlockSpec((1, OUT_CH), lambda i: (0, 0)),
        ],
        out_specs=pl.BlockSpec((_FB, OUT_CH), lambda i: (i, 0)),
        out_shape=jax.ShapeDtypeStruct((N, OUT_CH), jnp.float32),
    )(h, w2, b2)


def kernel(x, edge_index, lin1_w, lin1_b, conv_w, lin2_w, lin2_b):
    ei = edge_index.astype(jnp.int32)
    loops = jnp.arange(N, dtype=jnp.int32)
    src = jnp.concatenate([ei[0], loops])
    dst = jnp.concatenate([ei[1], loops])
    total = NC * NS * EPT
    padlen = total - src.shape[0]
    src = jnp.concatenate([src, jnp.zeros((padlen,), jnp.int32)])
    dst = jnp.concatenate([dst, jnp.full((padlen,), TRASH, jnp.int32)])
    src_idx = src.reshape(NC, NS, QC, G * CH)
    dst_idx = dst.reshape(NC, NS, QC, G * CH)

    beta = jnp.log(THETA / (jnp.arange(1, L_LAYERS + 1, dtype=jnp.float32)) + 1.0)
    wp_all = (beta[:, None, None] * conv_w
              + (1.0 - beta)[:, None, None] * jnp.eye(HID, dtype=jnp.float32))

    xp = jnp.pad(x, ((0, NPAD - N), (0, 0)))
    ones_u = jnp.ones((NPAD, HID), jnp.float32)
    pdeg = _sc_agg(ones_u, src_idx, dst_idx)
    h0, u0, dinv = _lin1(xp, lin1_w, lin1_b.reshape(1, HID), pdeg)

    def body(i, carry):
        u, _h = carry
        part = _sc_agg(u, src_idx, dst_idx)
        wp = lax.dynamic_index_in_dim(wp_all, i, keepdims=False)
        u, h = _tc_layer(part, h0, dinv, wp)
        return (u, h)

    _u, h = lax.fori_loop(0, L_LAYERS, body, (u0, h0))
    return _final(h, lin2_w, lin2_b.reshape(1, OUT_CH))


# speedup vs baseline: 4.1056x; 3.9740x over previous
import functools
import math

import jax
import jax.numpy as jnp
from jax import lax
from jax.experimental import pallas as pl
from jax.experimental.pallas import tpu as pltpu
from jax.experimental.pallas import tpu_sc as plsc

N = 10000
E = 320000
IN_CH = 128
HID = 64
OUT_CH = 128
L_LAYERS = 64
ALPHA = 0.5
THETA = 1.0

NC = 2
NS = 16
CH = 128
G = 4
NCHUNK = 84
QC = NCHUNK // G
EPT = NCHUNK * CH
NPAD = 10240
TRASH = N
ROWS_PT = NPAD // NS
ZB = ROWS_PT // CH


def _sc_agg_body(u_hbm, src_hbm, dst_hbm, out_hbm, src_v, dst_v, buf, agg_sh, sem):
    cid = lax.axis_index("c")
    sid = lax.axis_index("s")
    pltpu.sync_copy(src_hbm.at[cid, sid], src_v)
    pltpu.sync_copy(dst_hbm.at[cid, sid], dst_v)

    def _zrow(r, carry):
        def _zcol(k, c2):
            buf[r, pl.ds(k * 16, 16)] = jnp.zeros((16,), jnp.float32)
            return c2
        return lax.fori_loop(0, HID // 16, _zcol, carry)
    lax.fori_loop(0, CH, _zrow, 0)
    base = sid * ROWS_PT

    def _zcopy(k, carry):
        pltpu.sync_copy(buf.at[pl.ds(0, CH)], agg_sh.at[pl.ds(base + k * CH, CH)])
        return carry
    lax.fori_loop(0, ZB, _zcopy, 0)
    plsc.subcore_barrier()

    def _chunk(q, carry):
        pltpu.async_copy(u_hbm.at[src_v.at[q]], buf, sem).wait()
        pltpu.sync_copy(buf, agg_sh.at[dst_v.at[q]], add=True)
        return carry
    lax.fori_loop(0, QC, _chunk, 0)
    plsc.subcore_barrier()

    def _ocopy(k, carry):
        sl = pl.ds(base + k * CH, CH)
        pltpu.sync_copy(agg_sh.at[sl], out_hbm.at[cid, sl])
        return carry
    lax.fori_loop(0, ZB, _ocopy, 0)


@jax.jit
def _sc_agg(u, src_idx, dst_idx):
    return pl.kernel(
        _sc_agg_body,
        out_type=jax.ShapeDtypeStruct((NC, NPAD, HID), jnp.float32),
        mesh=plsc.VectorSubcoreMesh(core_axis_name="c", subcore_axis_name="s"),
        scratch_types=[
            pltpu.VMEM((QC, G * CH), jnp.int32),
            pltpu.VMEM((QC, G * CH), jnp.int32),
            pltpu.VMEM((G * CH, HID), jnp.float32),
            pltpu.VMEM_SHARED((NPAD, HID), jnp.float32),
            pltpu.SemaphoreType.DMA,
        ],
        compiler_params=pltpu.CompilerParams(use_tc_tiling_on_sc=False),
    )(u, src_idx, dst_idx)


def _lin1_body(x_ref, w1_ref, b1_ref, pdeg_ref, h0_ref, u0_ref, dinv_ref):
    deg = pdeg_ref[0, :, 0:1] + pdeg_ref[1, :, 0:1]
    dinv = jnp.where(deg > 0, lax.rsqrt(deg), 0.0)
    h0 = jnp.maximum(
        jnp.dot(x_ref[...], w1_ref[...], preferred_element_type=jnp.float32)
        + b1_ref[...], 0.0)
    h0_ref[...] = h0
    u0_ref[...] = dinv * h0
    dinv_ref[...] = dinv


def _tc_layer_body(p_ref, h0_ref, dinv_ref, wp_ref, u_ref, h_ref):
    dinv = dinv_ref[...]
    p = p_ref[0] + p_ref[1]
    t = (1.0 - ALPHA) * (dinv * p) + ALPHA * h0_ref[...]
    h = jnp.maximum(
        jnp.dot(t, wp_ref[...], preferred_element_type=jnp.float32), 0.0)
    h_ref[...] = h
    u_ref[...] = dinv * h


def _final_body(h_ref, w2_ref, b2_ref, o_ref):
    z = (jnp.dot(h_ref[...], w2_ref[...], preferred_element_type=jnp.float32)
         + b2_ref[...])
    m = jnp.max(z, axis=1, keepdims=True)
    lse = jnp.log(jnp.sum(jnp.exp(z - m), axis=1, keepdims=True)) + m
    o_ref[...] = z - lse


_RB = 2048


@jax.jit
def _lin1(xp, w1, b1, pdeg):
    grid = (NPAD // _RB,)
    return pl.pallas_call(
        _lin1_body,
        grid=grid,
        in_specs=[
            pl.BlockSpec((_RB, IN_CH), lambda i: (i, 0)),
            pl.BlockSpec((IN_CH, HID), lambda i: (0, 0)),
            pl.BlockSpec((1, HID), lambda i: (0, 0)),
            pl.BlockSpec((NC, _RB, HID), lambda i: (0, i, 0)),
        ],
        out_specs=[
            pl.BlockSpec((_RB, HID), lambda i: (i, 0)),
            pl.BlockSpec((_RB, HID), lambda i: (i, 0)),
            pl.BlockSpec((_RB, 1), lambda i: (i, 0)),
        ],
        out_shape=[
            jax.ShapeDtypeStruct((NPAD, HID), jnp.float32),
            jax.ShapeDtypeStruct((NPAD, HID), jnp.float32),
            jax.ShapeDtypeStruct((NPAD, 1), jnp.float32),
        ],
    )(xp, w1, b1, pdeg)


@jax.jit
def _tc_layer(part, h0, dinv, wp):
    grid = (NPAD // _RB,)
    return pl.pallas_call(
        _tc_layer_body,
        grid=grid,
        in_specs=[
            pl.BlockSpec((NC, _RB, HID), lambda i: (0, i, 0)),
            pl.BlockSpec((_RB, HID), lambda i: (i, 0)),
            pl.BlockSpec((_RB, 1), lambda i: (i, 0)),
            pl.BlockSpec((HID, HID), lambda i: (0, 0)),
        ],
        out_specs=[
            pl.BlockSpec((_RB, HID), lambda i: (i, 0)),
            pl.BlockSpec((_RB, HID), lambda i: (i, 0)),
        ],
        out_shape=[
            jax.ShapeDtypeStruct((NPAD, HID), jnp.float32),
            jax.ShapeDtypeStruct((NPAD, HID), jnp.float32),
        ],
    )(part, h0, dinv, wp)


_FB = 2000


@jax.jit
def _final(h, w2, b2):
    grid = (N // _FB,)
    return pl.pallas_call(
        _final_body,
        grid=grid,
        in_specs=[
            pl.BlockSpec((_FB, HID), lambda i: (i, 0)),
            pl.BlockSpec((HID, OUT_CH), lambda i: (0, 0)),
            pl.BlockSpec((1, OUT_CH), lambda i: (0, 0)),
        ],
        out_specs=pl.BlockSpec((_FB, OUT_CH), lambda i: (i, 0)),
        out_shape=jax.ShapeDtypeStruct((N, OUT_CH), jnp.float32),
    )(h, w2, b2)


def kernel(x, edge_index, lin1_w, lin1_b, conv_w, lin2_w, lin2_b):
    ei = edge_index.astype(jnp.int32)
    loops = jnp.arange(N, dtype=jnp.int32)
    src = jnp.concatenate([ei[0], loops])
    dst = jnp.concatenate([ei[1], loops])
    total = NC * NS * EPT
    padlen = total - src.shape[0]
    pad_ar = jnp.arange(padlen, dtype=jnp.int32)
    src = jnp.concatenate([src, pad_ar % N])
    dst = jnp.concatenate([dst, TRASH + pad_ar % (NPAD - N)])
    src_idx = src.reshape(NC, NS, QC, G * CH)
    dst_idx = dst.reshape(NC, NS, QC, G * CH)

    beta = jnp.log(THETA / (jnp.arange(1, L_LAYERS + 1, dtype=jnp.float32)) + 1.0)
    wp_all = (beta[:, None, None] * conv_w
              + (1.0 - beta)[:, None, None] * jnp.eye(HID, dtype=jnp.float32))

    xp = jnp.pad(x, ((0, NPAD - N), (0, 0)))
    ones_u = jnp.ones((NPAD, HID), jnp.float32)
    pdeg = _sc_agg(ones_u, src_idx, dst_idx)
    h0, u0, dinv = _lin1(xp, lin1_w, lin1_b.reshape(1, HID), pdeg)

    def body(i, carry):
        u, _h = carry
        part = _sc_agg(u, src_idx, dst_idx)
        wp = lax.dynamic_index_in_dim(wp_all, i, keepdims=False)
        u, h = _tc_layer(part, h0, dinv, wp)
        return (u, h)

    _u, h = lax.fori_loop(0, L_LAYERS, body, (u0, h0))
    return _final(h, lin2_w, lin2_b.reshape(1, OUT_CH))


# speedup vs baseline: 4.9790x; 1.2127x over previous
import functools
import math

import jax
import jax.numpy as jnp
from jax import lax
from jax.experimental import pallas as pl
from jax.experimental.pallas import tpu as pltpu
from jax.experimental.pallas import tpu_sc as plsc

N = 10000
E = 320000
IN_CH = 128
HID = 64
OUT_CH = 128
L_LAYERS = 64
ALPHA = 0.5
THETA = 1.0

NC = 2
NS = 16
CH = 128
G = 4
NCHUNK = 84
QC = NCHUNK // G
EPT = NCHUNK * CH
NPAD = 10240
TRASH = N
ROWS_PT = NPAD // NS
ZB = ROWS_PT // CH


def _sc_agg_body(u_hbm, src_hbm, dst_hbm, out_hbm, src_v, dst_v, buf, buf2,
                 agg_sh, sem, sem2):
    cid = lax.axis_index("c")
    sid = lax.axis_index("s")
    pltpu.sync_copy(src_hbm.at[cid, sid], src_v)
    pltpu.sync_copy(dst_hbm.at[cid, sid], dst_v)

    def _zrow(r, carry):
        def _zcol(k, c2):
            buf[r, pl.ds(k * 16, 16)] = jnp.zeros((16,), jnp.float32)
            return c2
        return lax.fori_loop(0, HID // 16, _zcol, carry)
    lax.fori_loop(0, CH, _zrow, 0)
    base = sid * ROWS_PT

    def _zcopy(k, carry):
        pltpu.sync_copy(buf.at[pl.ds(0, CH)], agg_sh.at[pl.ds(base + k * CH, CH)])
        return carry
    lax.fori_loop(0, ZB, _zcopy, 0)
    plsc.subcore_barrier()

    def _pair(p, carry):
        q0 = 2 * p
        pltpu.make_async_copy(u_hbm.at[src_v.at[q0]], buf, sem).wait()
        pltpu.async_copy(u_hbm.at[src_v.at[q0 + 1]], buf2, sem2)
        pltpu.sync_copy(buf, agg_sh.at[dst_v.at[q0]], add=True)
        pltpu.make_async_copy(u_hbm.at[src_v.at[q0 + 1]], buf2, sem2).wait()
        pltpu.async_copy(u_hbm.at[src_v.at[q0 + 2]], buf, sem)
        pltpu.sync_copy(buf2, agg_sh.at[dst_v.at[q0 + 1]], add=True)
        return carry
    pltpu.async_copy(u_hbm.at[src_v.at[0]], buf, sem)
    lax.fori_loop(0, (QC - 1) // 2, _pair, 0)
    pltpu.make_async_copy(u_hbm.at[src_v.at[QC - 1]], buf, sem).wait()
    pltpu.sync_copy(buf, agg_sh.at[dst_v.at[QC - 1]], add=True)
    plsc.subcore_barrier()

    def _ocopy(k, carry):
        sl = pl.ds(base + k * CH, CH)
        pltpu.sync_copy(agg_sh.at[sl], out_hbm.at[cid, sl])
        return carry
    lax.fori_loop(0, ZB, _ocopy, 0)


@jax.jit
def _sc_agg(u, src_idx, dst_idx):
    return pl.kernel(
        _sc_agg_body,
        out_type=jax.ShapeDtypeStruct((NC, NPAD, HID), jnp.float32),
        mesh=plsc.VectorSubcoreMesh(core_axis_name="c", subcore_axis_name="s"),
        scratch_types=[
            pltpu.VMEM((QC, G * CH), jnp.int32),
            pltpu.VMEM((QC, G * CH), jnp.int32),
            pltpu.VMEM((G * CH, HID), jnp.float32),
            pltpu.VMEM((G * CH, HID), jnp.float32),
            pltpu.VMEM_SHARED((NPAD, HID), jnp.float32),
            pltpu.SemaphoreType.DMA,
            pltpu.SemaphoreType.DMA,
        ],
        compiler_params=pltpu.CompilerParams(use_tc_tiling_on_sc=False),
    )(u, src_idx, dst_idx)


def _lin1_body(x_ref, w1_ref, b1_ref, pdeg_ref, h0_ref, u0_ref, dinv_ref):
    deg = pdeg_ref[0, :, 0:1] + pdeg_ref[1, :, 0:1]
    dinv = jnp.where(deg > 0, lax.rsqrt(deg), 0.0)
    h0 = jnp.maximum(
        jnp.dot(x_ref[...], w1_ref[...], preferred_element_type=jnp.float32)
        + b1_ref[...], 0.0)
    h0_ref[...] = h0
    u0_ref[...] = dinv * h0
    dinv_ref[...] = dinv


def _tc_layer_body(p_ref, h0_ref, dinv_ref, wp_ref, u_ref, h_ref):
    dinv = dinv_ref[...]
    p = p_ref[0] + p_ref[1]
    t = (1.0 - ALPHA) * (dinv * p) + ALPHA * h0_ref[...]
    h = jnp.maximum(
        jnp.dot(t, wp_ref[...], preferred_element_type=jnp.float32), 0.0)
    h_ref[...] = h
    u_ref[...] = dinv * h


def _final_body(h_ref, w2_ref, b2_ref, o_ref):
    z = (jnp.dot(h_ref[...], w2_ref[...], preferred_element_type=jnp.float32)
         + b2_ref[...])
    m = jnp.max(z, axis=1, keepdims=True)
    lse = jnp.log(jnp.sum(jnp.exp(z - m), axis=1, keepdims=True)) + m
    o_ref[...] = z - lse


_RB = 2048


@jax.jit
def _lin1(xp, w1, b1, pdeg):
    grid = (NPAD // _RB,)
    return pl.pallas_call(
        _lin1_body,
        grid=grid,
        in_specs=[
            pl.BlockSpec((_RB, IN_CH), lambda i: (i, 0)),
            pl.BlockSpec((IN_CH, HID), lambda i: (0, 0)),
            pl.BlockSpec((1, HID), lambda i: (0, 0)),
            pl.BlockSpec((NC, _RB, HID), lambda i: (0, i, 0)),
        ],
        out_specs=[
            pl.BlockSpec((_RB, HID), lambda i: (i, 0)),
            pl.BlockSpec((_RB, HID), lambda i: (i, 0)),
            pl.BlockSpec((_RB, 1), lambda i: (i, 0)),
        ],
        out_shape=[
            jax.ShapeDtypeStruct((NPAD, HID), jnp.float32),
            jax.ShapeDtypeStruct((NPAD, HID), jnp.float32),
            jax.ShapeDtypeStruct((NPAD, 1), jnp.float32),
        ],
    )(xp, w1, b1, pdeg)


@jax.jit
def _tc_layer(part, h0, dinv, wp):
    grid = (NPAD // _RB,)
    return pl.pallas_call(
        _tc_layer_body,
        grid=grid,
        in_specs=[
            pl.BlockSpec((NC, _RB, HID), lambda i: (0, i, 0)),
            pl.BlockSpec((_RB, HID), lambda i: (i, 0)),
            pl.BlockSpec((_RB, 1), lambda i: (i, 0)),
            pl.BlockSpec((HID, HID), lambda i: (0, 0)),
        ],
        out_specs=[
            pl.BlockSpec((_RB, HID), lambda i: (i, 0)),
            pl.BlockSpec((_RB, HID), lambda i: (i, 0)),
        ],
        out_shape=[
            jax.ShapeDtypeStruct((NPAD, HID), jnp.float32),
            jax.ShapeDtypeStruct((NPAD, HID), jnp.float32),
        ],
    )(part, h0, dinv, wp)


_FB = 2000


@jax.jit
def _final(h, w2, b2):
    grid = (N // _FB,)
    return pl.pallas_call(
        _final_body,
        grid=grid,
        in_specs=[
            pl.BlockSpec((_FB, HID), lambda i: (i, 0)),
            pl.BlockSpec((HID, OUT_CH), lambda i: (0, 0)),
            pl.BlockSpec((1, OUT_CH), lambda i: (0, 0)),
        ],
        out_specs=pl.BlockSpec((_FB, OUT_CH), lambda i: (i, 0)),
        out_shape=jax.ShapeDtypeStruct((N, OUT_CH), jnp.float32),
    )(h, w2, b2)


def kernel(x, edge_index, lin1_w, lin1_b, conv_w, lin2_w, lin2_b):
    ei = edge_index.astype(jnp.int32)
    loops = jnp.arange(N, dtype=jnp.int32)
    src = jnp.concatenate([ei[0], loops])
    dst = jnp.concatenate([ei[1], loops])
    total = NC * NS * EPT
    padlen = total - src.shape[0]
    pad_ar = jnp.arange(padlen, dtype=jnp.int32)
    src = jnp.concatenate([src, pad_ar % N])
    dst = jnp.concatenate([dst, TRASH + pad_ar % (NPAD - N)])
    src_idx = src.reshape(NC, NS, QC, G * CH)
    dst_idx = dst.reshape(NC, NS, QC, G * CH)

    beta = jnp.log(THETA / (jnp.arange(1, L_LAYERS + 1, dtype=jnp.float32)) + 1.0)
    wp_all = (beta[:, None, None] * conv_w
              + (1.0 - beta)[:, None, None] * jnp.eye(HID, dtype=jnp.float32))

    xp = jnp.pad(x, ((0, NPAD - N), (0, 0)))
    ones_u = jnp.ones((NPAD, HID), jnp.float32)
    pdeg = _sc_agg(ones_u, src_idx, dst_idx)
    h0, u0, dinv = _lin1(xp, lin1_w, lin1_b.reshape(1, HID), pdeg)

    def body(i, carry):
        u, _h = carry
        part = _sc_agg(u, src_idx, dst_idx)
        wp = lax.dynamic_index_in_dim(wp_all, i, keepdims=False)
        u, h = _tc_layer(part, h0, dinv, wp)
        return (u, h)

    _u, h = lax.fori_loop(0, L_LAYERS, body, (u0, h0))
    return _final(h, lin2_w, lin2_b.reshape(1, OUT_CH))


# speedup vs baseline: 5.7268x; 1.1502x over previous
import functools
import math

import jax
import jax.numpy as jnp
from jax import lax
from jax.experimental import pallas as pl
from jax.experimental.pallas import tpu as pltpu
from jax.experimental.pallas import tpu_sc as plsc

N = 10000
E = 320000
IN_CH = 128
HID = 64
OUT_CH = 128
L_LAYERS = 64
ALPHA = 0.5
THETA = 1.0

NC = 2
NS = 16
CH = 128
G = 2
NCHUNK = 84
QC = NCHUNK // G
EPT = NCHUNK * CH
NPAD = 10240
TRASH = N
ROWS_PT = NPAD // NS
ZB = ROWS_PT // CH


def _sc_agg_body(u_hbm, src_hbm, dst_hbm, out_hbm, src_v, dst_v,
                 b0, b1, b2, agg_sh, s0, s1, s2):
    bufs = (b0, b1, b2)
    sems = (s0, s1, s2)
    cid = lax.axis_index("c")
    sid = lax.axis_index("s")
    pltpu.sync_copy(src_hbm.at[cid, sid], src_v)
    pltpu.sync_copy(dst_hbm.at[cid, sid], dst_v)

    pltpu.async_copy(u_hbm.at[src_v.at[0]], b0, s0)
    pltpu.async_copy(u_hbm.at[src_v.at[1]], b1, s1)

    def _zrow(r, carry):
        def _zcol(k, c2):
            b2[r, pl.ds(k * 16, 16)] = jnp.zeros((16,), jnp.float32)
            return c2
        return lax.fori_loop(0, HID // 16, _zcol, carry)
    lax.fori_loop(0, CH, _zrow, 0)
    base = sid * ROWS_PT

    def _zcopy(k, carry):
        pltpu.sync_copy(b2.at[pl.ds(0, CH)], agg_sh.at[pl.ds(base + k * CH, CH)])
        return carry
    lax.fori_loop(0, ZB, _zcopy, 0)
    plsc.subcore_barrier()

    def _triple(p, carry):
        for b in range(3):
            c = 3 * p + b
            pltpu.make_async_copy(u_hbm.at[src_v.at[c]], bufs[b], sems[b]).wait()

            @pl.when(c + 2 < QC)
            def _():
                nb = (b + 2) % 3
                pltpu.async_copy(u_hbm.at[src_v.at[c + 2]], bufs[nb], sems[nb])
            pltpu.sync_copy(bufs[b], agg_sh.at[dst_v.at[c]], add=True)
        return carry
    lax.fori_loop(0, QC // 3, _triple, 0)
    plsc.subcore_barrier()

    def _ocopy(k, carry):
        sl = pl.ds(base + k * CH, CH)
        pltpu.sync_copy(agg_sh.at[sl], out_hbm.at[cid, sl])
        return carry
    lax.fori_loop(0, ZB, _ocopy, 0)


@jax.jit
def _sc_agg(u, src_idx, dst_idx):
    return pl.kernel(
        _sc_agg_body,
        out_type=jax.ShapeDtypeStruct((NC, NPAD, HID), jnp.float32),
        mesh=plsc.VectorSubcoreMesh(core_axis_name="c", subcore_axis_name="s"),
        scratch_types=[
            pltpu.VMEM((QC, G * CH), jnp.int32),
            pltpu.VMEM((QC, G * CH), jnp.int32),
            pltpu.VMEM((G * CH, HID), jnp.float32),
            pltpu.VMEM((G * CH, HID), jnp.float32),
            pltpu.VMEM((G * CH, HID), jnp.float32),
            pltpu.VMEM_SHARED((NPAD, HID), jnp.float32),
            pltpu.SemaphoreType.DMA,
            pltpu.SemaphoreType.DMA,
            pltpu.SemaphoreType.DMA,
        ],
        compiler_params=pltpu.CompilerParams(use_tc_tiling_on_sc=False),
    )(u, src_idx, dst_idx)


def _lin1_body(x_ref, w1_ref, b1_ref, pdeg_ref, h0_ref, u0_ref, dinv_ref):
    deg = pdeg_ref[0, :, 0:1] + pdeg_ref[1, :, 0:1]
    dinv = jnp.where(deg > 0, lax.rsqrt(deg), 0.0)
    h0 = jnp.maximum(
        jnp.dot(x_ref[...], w1_ref[...], preferred_element_type=jnp.float32)
        + b1_ref[...], 0.0)
    h0_ref[...] = h0
    u0_ref[...] = dinv * h0
    dinv_ref[...] = dinv


def _tc_layer_body(p_ref, h0_ref, dinv_ref, wp_ref, u_ref, h_ref):
    dinv = dinv_ref[...]
    p = p_ref[0] + p_ref[1]
    t = (1.0 - ALPHA) * (dinv * p) + ALPHA * h0_ref[...]
    h = jnp.maximum(
        jnp.dot(t, wp_ref[...], preferred_element_type=jnp.float32), 0.0)
    h_ref[...] = h
    u_ref[...] = dinv * h


def _final_body(h_ref, w2_ref, b2_ref, o_ref):
    z = (jnp.dot(h_ref[...], w2_ref[...], preferred_element_type=jnp.float32)
         + b2_ref[...])
    m = jnp.max(z, axis=1, keepdims=True)
    lse = jnp.log(jnp.sum(jnp.exp(z - m), axis=1, keepdims=True)) + m
    o_ref[...] = z - lse


_RB = 2048


@jax.jit
def _lin1(xp, w1, b1, pdeg):
    grid = (NPAD // _RB,)
    return pl.pallas_call(
        _lin1_body,
        grid=grid,
        in_specs=[
            pl.BlockSpec((_RB, IN_CH), lambda i: (i, 0)),
            pl.BlockSpec((IN_CH, HID), lambda i: (0, 0)),
            pl.BlockSpec((1, HID), lambda i: (0, 0)),
            pl.BlockSpec((NC, _RB, HID), lambda i: (0, i, 0)),
        ],
        out_specs=[
            pl.BlockSpec((_RB, HID), lambda i: (i, 0)),
            pl.BlockSpec((_RB, HID), lambda i: (i, 0)),
            pl.BlockSpec((_RB, 1), lambda i: (i, 0)),
        ],
        out_shape=[
            jax.ShapeDtypeStruct((NPAD, HID), jnp.float32),
            jax.ShapeDtypeStruct((NPAD, HID), jnp.float32),
            jax.ShapeDtypeStruct((NPAD, 1), jnp.float32),
        ],
    )(xp, w1, b1, pdeg)


@jax.jit
def _tc_layer(part, h0, dinv, wp):
    grid = (NPAD // _RB,)
    return pl.pallas_call(
        _tc_layer_body,
        grid=grid,
        in_specs=[
            pl.BlockSpec((NC, _RB, HID), lambda i: (0, i, 0)),
            pl.BlockSpec((_RB, HID), lambda i: (i, 0)),
            pl.BlockSpec((_RB, 1), lambda i: (i, 0)),
            pl.BlockSpec((HID, HID), lambda i: (0, 0)),
        ],
        out_specs=[
            pl.BlockSpec((_RB, HID), lambda i: (i, 0)),
            pl.BlockSpec((_RB, HID), lambda i: (i, 0)),
        ],
        out_shape=[
            jax.ShapeDtypeStruct((NPAD, HID), jnp.float32),
            jax.ShapeDtypeStruct((NPAD, HID), jnp.float32),
        ],
    )(part, h0, dinv, wp)


_FB = 2000


@jax.jit
def _final(h, w2, b2):
    grid = (N // _FB,)
    return pl.pallas_call(
        _final_body,
        grid=grid,
        in_specs=[
            pl.BlockSpec((_FB, HID), lambda i: (i, 0)),
            pl.BlockSpec((HID, OUT_CH), lambda i: (0, 0)),
            pl.BlockSpec((1, OUT_CH), lambda i: (0, 0)),
        ],
        out_specs=pl.BlockSpec((_FB, OUT_CH), lambda i: (i, 0)),
        out_shape=jax.ShapeDtypeStruct((N, OUT_CH), jnp.float32),
    )(h, w2, b2)


def kernel(x, edge_index, lin1_w, lin1_b, conv_w, lin2_w, lin2_b):
    ei = edge_index.astype(jnp.int32)
    loops = jnp.arange(N, dtype=jnp.int32)
    src = jnp.concatenate([ei[0], loops])
    dst = jnp.concatenate([ei[1], loops])
    total = NC * NS * EPT
    padlen = total - src.shape[0]
    pad_ar = jnp.arange(padlen, dtype=jnp.int32)
    src = jnp.concatenate([src, pad_ar % N])
    dst = jnp.concatenate([dst, TRASH + pad_ar % (NPAD - N)])
    src_idx = src.reshape(NC, NS, QC, G * CH)
    dst_idx = dst.reshape(NC, NS, QC, G * CH)

    beta = jnp.log(THETA / (jnp.arange(1, L_LAYERS + 1, dtype=jnp.float32)) + 1.0)
    wp_all = (beta[:, None, None] * conv_w
              + (1.0 - beta)[:, None, None] * jnp.eye(HID, dtype=jnp.float32))

    xp = jnp.pad(x, ((0, NPAD - N), (0, 0)))
    ones_u = jnp.ones((NPAD, HID), jnp.float32)
    pdeg = _sc_agg(ones_u, src_idx, dst_idx)
    h0, u0, dinv = _lin1(xp, lin1_w, lin1_b.reshape(1, HID), pdeg)

    def body(i, carry):
        u, _h = carry
        part = _sc_agg(u, src_idx, dst_idx)
        wp = lax.dynamic_index_in_dim(wp_all, i, keepdims=False)
        u, h = _tc_layer(part, h0, dinv, wp)
        return (u, h)

    _u, h = lax.fori_loop(0, L_LAYERS, body, (u0, h0))
    return _final(h, lin2_w, lin2_b.reshape(1, OUT_CH))
